# baseline (speedup 1.0000x reference)
import functools

import jax
import jax.numpy as jnp
from jax.experimental import pallas as pl
from jax.experimental.pallas import tpu as pltpu

PAD_ID = 0
NTP = 8
VPAD = 56
VPAD2 = 64
EMB = 16
CH = 32
KMAX = 5
H = 32
STOP = 5
BB = 256

N_EMB = 0
N_WCONV = VPAD
N_WFC = N_WCONV + KMAX * EMB
N_TRANS = N_WFC + 2 * H
N_BCONV = N_TRANS + NTP
N_BFC = N_BCONV + 8
N_BOS = N_BFC + 8
W_WX = 0
W_WH = CH
W_BX = W_WH + 2 * H


def _fused(xid_ref, xrow_ref, wemb_ref, wconv_ref, wx_ref, wh_ref, wfc_ref,
           crf_ref, path_ref, score_ref, hid_ref, *, T):
    TB = T * BB
    f32 = jnp.float32
    i32 = jnp.int32

    ids2d = xid_ref[...]
    idsrow = xrow_ref[...]
    maskb = ids2d != PAD_ID

    oh = (jax.lax.broadcasted_iota(i32, (VPAD2, TB), 0) == idsrow).astype(f32)
    embT = jnp.dot(wemb_ref[...], oh, preferred_element_type=f32)

    bconv_b = crf_ref[80:112, :]
    bfc_b = crf_ref[112:120, :]
    bx_b = crf_ref[120:376, :]

    parts = []
    for k in range(KMAX):
        s = k - KMAX // 2
        if s == 0:
            parts.append(embT)
        elif s > 0:
            parts.append(jnp.concatenate(
                [embT[:, s * BB:], jnp.zeros((EMB, s * BB), f32)], axis=1))
        else:
            parts.append(jnp.concatenate(
                [jnp.zeros((EMB, -s * BB), f32), embT[:, :TB + s * BB]], axis=1))
    colsT = jnp.concatenate(parts, axis=0)
    convT = jnp.maximum(
        jnp.dot(wconv_ref[...], colsT, preferred_element_type=f32)
        + jnp.tile(bconv_b, (1, T)), 0.0)

    gxT = jnp.dot(wx_ref[...], convT, preferred_element_type=f32)

    whf = wh_ref[0:4 * H, :]
    whb = wh_ref[4 * H:8 * H, :]
    bxf = bx_b[0:4 * H, :]
    bxb = bx_b[4 * H:8 * H, :]

    hf = jnp.zeros((H, BB), f32)
    cf = jnp.zeros((H, BB), f32)
    hb = jnp.zeros((H, BB), f32)
    cb = jnp.zeros((H, BB), f32)
    for step in range(T):
        tf, tb = step, T - 1 - step
        gf = (gxT[0:4 * H, tf * BB:(tf + 1) * BB] + bxf
              ) + jnp.dot(whf, hf, preferred_element_type=f32)
        gb = (gxT[4 * H:8 * H, tb * BB:(tb + 1) * BB] + bxb
              ) + jnp.dot(whb, hb, preferred_element_type=f32)
        sf = jax.nn.sigmoid(gf[0:3 * H, :])
        sb = jax.nn.sigmoid(gb[0:3 * H, :])
        cgf = jnp.tanh(gf[3 * H:4 * H, :])
        cgb = jnp.tanh(gb[3 * H:4 * H, :])
        cf_new = sf[H:2 * H, :] * cf + sf[0:H, :] * cgf
        cb_new = sb[H:2 * H, :] * cb + sb[0:H, :] * cgb
        hf_new = sf[2 * H:3 * H, :] * jnp.tanh(cf_new)
        hb_new = sb[2 * H:3 * H, :] * jnp.tanh(cb_new)
        keep_f = jnp.broadcast_to(maskb[tf:tf + 1, :], (H, BB))
        keep_b = jnp.broadcast_to(maskb[tb:tb + 1, :], (H, BB))
        hf = jnp.where(keep_f, hf_new, hf)
        cf = jnp.where(keep_f, cf_new, cf)
        hb = jnp.where(keep_b, hb_new, hb)
        cb = jnp.where(keep_b, cb_new, cb)
        hid_ref[0:H, tf * BB:(tf + 1) * BB] = hf
        hid_ref[H:2 * H, tb * BB:(tb + 1) * BB] = hb

    emT = jnp.dot(wfc_ref[...], hid_ref[...], preferred_element_type=f32)

    trans3 = crf_ref[0:64, :].reshape(NTP, NTP, BB)
    bos_b = crf_ref[64:72, :]
    stop_b = crf_ref[72:80, :]
    alphas = bos_b + (emT[:, 0:BB] + bfc_b)
    iota_prev = jax.lax.broadcasted_iota(i32, (NTP, NTP, BB), 1)
    iota_tag = jax.lax.broadcasted_iota(i32, (NTP, BB), 0)
    bp = [None] * (T - 1)
    for i in range(1, T):
        scores = trans3 + alphas[None, :, :]
        mx = jnp.max(scores, axis=1)
        bp[i - 1] = jnp.min(
            jnp.where(scores >= mx[:, None, :], iota_prev, NTP), axis=1)
        alphas = jnp.where(maskb[i:i + 1, :],
                           mx + (emT[:, i * BB:(i + 1) * BB] + bfc_b), alphas)

    end_scores = alphas + stop_b
    best = jnp.max(end_scores, axis=0, keepdims=True)
    final_tag = jnp.min(
        jnp.where(end_scores >= best, iota_tag, NTP), axis=0, keepdims=True)

    lengths = jnp.sum(maskb.astype(i32), axis=0, keepdims=True)
    cur = final_tag
    rows = [None] * T
    for t in range(T - 1, -1, -1):
        bp_t = bp[min(t, T - 2)]
        prev = jnp.sum(jnp.where(iota_tag == cur, bp_t, 0), axis=0, keepdims=True)
        cur = jnp.where((lengths - 1) == t, final_tag,
                        jnp.where(t < lengths - 1, prev, cur))
        rows[t] = jnp.where(t < lengths, cur, PAD_ID)

    path_ref[...] = jnp.concatenate(rows, axis=0)
    score_ref[...] = best


def kernel(narrow, wide, x):
    B0, T = x.shape
    BP = ((B0 + BB - 1) // BB) * BB
    nb = BP // BB
    TB = T * BB

    wemb = jnp.zeros((EMB, VPAD2), jnp.float32).at[:, :VPAD].set(
        narrow[N_EMB:N_EMB + VPAD, 0:EMB].T)
    wconv = narrow[N_WCONV:N_WCONV + KMAX * EMB, 0:CH].T

    def sel(m, d):
        return jnp.concatenate(
            [m[:, (2 * g + d) * H:(2 * g + d + 1) * H] for g in range(4)], axis=1)

    wx_m = wide[W_WX:W_WX + CH, :]
    wx = jnp.concatenate([sel(wx_m, 0), sel(wx_m, 1)], axis=1).T
    wh = jnp.concatenate(
        [sel(wide[W_WH:W_WH + H, :], 0),
         sel(wide[W_WH + H:W_WH + 2 * H, :], 1)], axis=1).T
    bx_m = wide[W_BX, :][None, :]
    bx = jnp.concatenate([sel(bx_m, 0), sel(bx_m, 1)], axis=1)[0]
    wfc = narrow[N_WFC:N_WFC + 2 * H, 0:NTP].T
    transT = narrow[N_TRANS:N_TRANS + NTP, 0:NTP]
    crf = jnp.concatenate(
        [jnp.broadcast_to(transT.reshape(64, 1), (64, BB)),
         jnp.broadcast_to(narrow[N_BOS, 0:NTP][:, None], (NTP, BB)),
         jnp.broadcast_to(transT[STOP, :][:, None], (NTP, BB)),
         jnp.broadcast_to(narrow[N_BCONV, 0:CH][:, None], (CH, BB)),
         jnp.broadcast_to(narrow[N_BFC, 0:NTP][:, None], (NTP, BB)),
         jnp.broadcast_to(bx[:, None], (8 * H, BB))], axis=0)

    xp = jnp.pad(x.astype(jnp.int32), ((0, BP - B0), (0, 0)),
                 constant_values=PAD_ID)
    xT3 = xp.reshape(nb, BB, T).transpose(0, 2, 1)
    xrow3 = xT3.reshape(nb, 1, TB)

    path3, score3 = pl.pallas_call(
        functools.partial(_fused, T=T),
        grid=(nb,),
        in_specs=[
            pl.BlockSpec((None, T, BB), lambda j: (j, 0, 0)),
            pl.BlockSpec((None, 1, TB), lambda j: (j, 0, 0)),
            pl.BlockSpec((EMB, VPAD2), lambda j: (0, 0)),
            pl.BlockSpec((CH, KMAX * EMB), lambda j: (0, 0)),
            pl.BlockSpec((8 * H, CH), lambda j: (0, 0)),
            pl.BlockSpec((8 * H, H), lambda j: (0, 0)),
            pl.BlockSpec((NTP, 2 * H), lambda j: (0, 0)),
            pl.BlockSpec((376, BB), lambda j: (0, 0)),
        ],
        out_specs=[
            pl.BlockSpec((None, T, BB), lambda j: (j, 0, 0)),
            pl.BlockSpec((None, 1, BB), lambda j: (j, 0, 0)),
        ],
        out_shape=[
            jax.ShapeDtypeStruct((nb, T, BB), jnp.int32),
            jax.ShapeDtypeStruct((nb, 1, BB), jnp.float32),
        ],
        scratch_shapes=[pltpu.VMEM((2 * H, TB), jnp.float32)],
        compiler_params=pltpu.CompilerParams(dimension_semantics=("parallel",)),
    )(xT3, xrow3, wemb, wconv, wx, wh, wfc, crf)

    path = path3.transpose(0, 2, 1).reshape(BP, T)[:B0]
    score = score3.reshape(BP)[:B0]
    return score, path


# speedup vs baseline: 1.1265x; 1.1265x over previous
import functools

import jax
import jax.numpy as jnp
from jax.experimental import pallas as pl
from jax.experimental.pallas import tpu as pltpu

PAD_ID = 0
NTP = 8
VPAD = 56
VPAD2 = 64
EMB = 16
CH = 32
KMAX = 5
H = 32
STOP = 5
BB = 256

N_EMB = 0
N_WCONV = VPAD
N_WFC = N_WCONV + KMAX * EMB
N_TRANS = N_WFC + 2 * H
N_BCONV = N_TRANS + NTP
N_BFC = N_BCONV + 8
N_BOS = N_BFC + 8
W_WX = 0
W_WH = CH
W_BX = W_WH + 2 * H


def _fused(xid_ref, xrow_ref, wemb_ref, wconv_ref, wx_ref, wh_ref, wfc_ref,
           crf_ref, path_ref, score_ref, hid_ref, *, T):
    TB = T * BB
    f32 = jnp.float32
    i32 = jnp.int32

    ids2d = xid_ref[...]
    idsrow = xrow_ref[...]
    maskb = ids2d != PAD_ID

    oh = (jax.lax.broadcasted_iota(i32, (VPAD2, TB), 0) == idsrow).astype(f32)
    embT = jnp.dot(wemb_ref[...], oh, preferred_element_type=f32)

    bconv_b = crf_ref[80:112, :]
    bfc_b = crf_ref[112:120, :]
    bx_b = crf_ref[120:376, :]

    parts = []
    for k in range(KMAX):
        s = k - KMAX // 2
        if s == 0:
            parts.append(embT)
        elif s > 0:
            parts.append(jnp.concatenate(
                [embT[:, s * BB:], jnp.zeros((EMB, s * BB), f32)], axis=1))
        else:
            parts.append(jnp.concatenate(
                [jnp.zeros((EMB, -s * BB), f32), embT[:, :TB + s * BB]], axis=1))
    colsT = jnp.concatenate(parts, axis=0)
    convT = jnp.maximum(
        jnp.dot(wconv_ref[...], colsT, preferred_element_type=f32)
        + jnp.tile(bconv_b, (1, T)), 0.0)

    gxT = jnp.dot(wx_ref[...], convT, preferred_element_type=f32)

    wh = wh_ref[...]

    hf = jnp.zeros((H, BB), f32)
    cf = jnp.zeros((H, BB), f32)
    hb = jnp.zeros((H, BB), f32)
    cb = jnp.zeros((H, BB), f32)
    for step in range(T):
        tf, tb = step, T - 1 - step
        lf = slice(tf * BB, (tf + 1) * BB)
        lb = slice(tb * BB, (tb + 1) * BB)
        gin = jnp.concatenate(
            [gxT[0 * H:1 * H, lf], gxT[1 * H:2 * H, lb],
             gxT[2 * H:3 * H, lf], gxT[3 * H:4 * H, lb],
             gxT[4 * H:5 * H, lf], gxT[5 * H:6 * H, lb],
             gxT[6 * H:7 * H, lf], gxT[7 * H:8 * H, lb]], axis=0)
        g = (gin + bx_b) + jnp.dot(
            wh, jnp.concatenate([hf, hb], axis=0), preferred_element_type=f32)
        sig = jax.nn.sigmoid(g[0:6 * H, :])
        cg = jnp.tanh(g[6 * H:8 * H, :])
        cf = sig[2 * H:3 * H, :] * cf + sig[0:H, :] * cg[0:H, :]
        hf = sig[4 * H:5 * H, :] * jnp.tanh(cf)
        cb_new = sig[3 * H:4 * H, :] * cb + sig[H:2 * H, :] * cg[H:2 * H, :]
        hb_new = sig[5 * H:6 * H, :] * jnp.tanh(cb_new)
        keep_b = jnp.broadcast_to(maskb[tb:tb + 1, :], (H, BB))
        hb = jnp.where(keep_b, hb_new, hb)
        cb = jnp.where(keep_b, cb_new, cb)
        hid_ref[0:H, lf] = hf
        hid_ref[H:2 * H, lb] = hb

    emT = jnp.dot(wfc_ref[...], hid_ref[...], preferred_element_type=f32)

    trans3 = crf_ref[0:64, :].reshape(NTP, NTP, BB)
    bos_b = crf_ref[64:72, :]
    stop_b = crf_ref[72:80, :]
    alphas = bos_b + (emT[:, 0:BB] + bfc_b)
    iota_prev = jax.lax.broadcasted_iota(i32, (NTP, NTP, BB), 1)
    iota_tag = jax.lax.broadcasted_iota(i32, (NTP, BB), 0)
    bp = [None] * (T - 1)
    for i in range(1, T):
        scores = trans3 + alphas[None, :, :]
        mx = jnp.max(scores, axis=1)
        bp[i - 1] = jnp.min(
            jnp.where(scores >= mx[:, None, :], iota_prev, NTP), axis=1)
        alphas = jnp.where(maskb[i:i + 1, :],
                           mx + (emT[:, i * BB:(i + 1) * BB] + bfc_b), alphas)

    end_scores = alphas + stop_b
    best = jnp.max(end_scores, axis=0, keepdims=True)
    final_tag = jnp.min(
        jnp.where(end_scores >= best, iota_tag, NTP), axis=0, keepdims=True)

    lengths = jnp.sum(maskb.astype(i32), axis=0, keepdims=True)
    cur = final_tag
    rows = [None] * T
    for t in range(T - 1, -1, -1):
        bp_t = bp[min(t, T - 2)]
        prev = jnp.sum(jnp.where(iota_tag == cur, bp_t, 0), axis=0, keepdims=True)
        cur = jnp.where((lengths - 1) == t, final_tag,
                        jnp.where(t < lengths - 1, prev, cur))
        rows[t] = jnp.where(t < lengths, cur, PAD_ID)

    path_ref[...] = jnp.concatenate(rows, axis=0)
    score_ref[...] = best


def kernel(narrow, wide, x):
    B0, T = x.shape
    BP = ((B0 + BB - 1) // BB) * BB
    nb = BP // BB
    TB = T * BB

    wemb = jnp.zeros((EMB, VPAD2), jnp.float32).at[:, :VPAD].set(
        narrow[N_EMB:N_EMB + VPAD, 0:EMB].T)
    wconv = narrow[N_WCONV:N_WCONV + KMAX * EMB, 0:CH].T
    wx = wide[W_WX:W_WX + CH, :].T
    wh = wide[W_WH:W_WH + 2 * H, :].T
    wfc = narrow[N_WFC:N_WFC + 2 * H, 0:NTP].T
    transT = narrow[N_TRANS:N_TRANS + NTP, 0:NTP]
    crf = jnp.concatenate(
        [jnp.broadcast_to(transT.reshape(64, 1), (64, BB)),
         jnp.broadcast_to(narrow[N_BOS, 0:NTP][:, None], (NTP, BB)),
         jnp.broadcast_to(transT[STOP, :][:, None], (NTP, BB)),
         jnp.broadcast_to(narrow[N_BCONV, 0:CH][:, None], (CH, BB)),
         jnp.broadcast_to(narrow[N_BFC, 0:NTP][:, None], (NTP, BB)),
         jnp.broadcast_to(wide[W_BX, :][:, None], (8 * H, BB))], axis=0)

    xp = jnp.pad(x.astype(jnp.int32), ((0, BP - B0), (0, 0)),
                 constant_values=PAD_ID)
    xT3 = xp.reshape(nb, BB, T).transpose(0, 2, 1)
    xrow3 = xT3.reshape(nb, 1, TB)

    path3, score3 = pl.pallas_call(
        functools.partial(_fused, T=T),
        grid=(nb,),
        in_specs=[
            pl.BlockSpec((None, T, BB), lambda j: (j, 0, 0)),
            pl.BlockSpec((None, 1, TB), lambda j: (j, 0, 0)),
            pl.BlockSpec((EMB, VPAD2), lambda j: (0, 0)),
            pl.BlockSpec((CH, KMAX * EMB), lambda j: (0, 0)),
            pl.BlockSpec((8 * H, CH), lambda j: (0, 0)),
            pl.BlockSpec((8 * H, 2 * H), lambda j: (0, 0)),
            pl.BlockSpec((NTP, 2 * H), lambda j: (0, 0)),
            pl.BlockSpec((376, BB), lambda j: (0, 0)),
        ],
        out_specs=[
            pl.BlockSpec((None, T, BB), lambda j: (j, 0, 0)),
            pl.BlockSpec((None, 1, BB), lambda j: (j, 0, 0)),
        ],
        out_shape=[
            jax.ShapeDtypeStruct((nb, T, BB), jnp.int32),
            jax.ShapeDtypeStruct((nb, 1, BB), jnp.float32),
        ],
        scratch_shapes=[pltpu.VMEM((2 * H, TB), jnp.float32)],
        compiler_params=pltpu.CompilerParams(dimension_semantics=("parallel",)),
    )(xT3, xrow3, wemb, wconv, wx, wh, wfc, crf)

    path = path3.transpose(0, 2, 1).reshape(BP, T)[:B0]
    score = score3.reshape(BP)[:B0]
    return score, path


# speedup vs baseline: 1.2897x; 1.1449x over previous
import functools

import jax
import jax.numpy as jnp
from jax.experimental import pallas as pl
from jax.experimental.pallas import tpu as pltpu

PAD_ID = 0
NTP = 8
VPAD = 56
VPAD2 = 64
EMB = 16
CH = 32
KMAX = 5
H = 32
STOP = 5
BB = 256

N_EMB = 0
N_WCONV = VPAD
N_WFC = N_WCONV + KMAX * EMB
N_TRANS = N_WFC + 2 * H
N_BCONV = N_TRANS + NTP
N_BFC = N_BCONV + 8
N_BOS = N_BFC + 8
W_WX = 0
W_WH = CH
W_BX = W_WH + 2 * H


def _fused(xid_ref, xrow_ref, wemb_ref, wconv_ref, wx_ref, wh_ref, wfc_ref,
           crf_ref, path_ref, score_ref, hid_ref, *, T):
    TB = T * BB
    f32 = jnp.float32
    i32 = jnp.int32

    ids2d = xid_ref[...]
    idsrow = xrow_ref[...]
    maskb = ids2d != PAD_ID
    mrowH = jnp.broadcast_to(idsrow != PAD_ID, (H, TB))
    mrow8 = mrowH[0:NTP, :]

    oh = (jax.lax.broadcasted_iota(i32, (VPAD2, TB), 0) == idsrow).astype(f32)
    embT = jnp.dot(wemb_ref[...], oh, preferred_element_type=f32)

    bconv_b = crf_ref[80:112, :]
    bfc_b = crf_ref[112:120, :]
    bx_b = crf_ref[120:376, :]

    parts = []
    for k in range(KMAX):
        s = k - KMAX // 2
        if s == 0:
            parts.append(embT)
        elif s > 0:
            parts.append(jnp.concatenate(
                [embT[:, s * BB:], jnp.zeros((EMB, s * BB), f32)], axis=1))
        else:
            parts.append(jnp.concatenate(
                [jnp.zeros((EMB, -s * BB), f32), embT[:, :TB + s * BB]], axis=1))
    colsT = jnp.concatenate(parts, axis=0)
    convT = jnp.maximum(
        jnp.dot(wconv_ref[...], colsT, preferred_element_type=f32)
        + jnp.tile(bconv_b, (1, T)), 0.0)

    gxT = jnp.dot(wx_ref[...], convT, preferred_element_type=f32)

    frow = ((jax.lax.broadcasted_iota(i32, (8 * H, BB), 0) // H) % 2) == 0
    wh = wh_ref[...]

    h = jnp.zeros((2 * H, BB), f32)
    c = jnp.zeros((2 * H, BB), f32)
    for step in range(T):
        tf, tb = step, T - 1 - step
        g_tf = gxT[:, tf * BB:(tf + 1) * BB]
        g_tb = gxT[:, tb * BB:(tb + 1) * BB]
        g = (jnp.where(frow, g_tf, g_tb) + bx_b
             ) + jnp.dot(wh, h, preferred_element_type=f32)
        sig = jax.nn.sigmoid(g[0:6 * H, :])
        cg = jnp.tanh(g[6 * H:8 * H, :])
        c_new = sig[2 * H:4 * H, :] * c + sig[0:2 * H, :] * cg
        h_new = sig[4 * H:6 * H, :] * jnp.tanh(c_new)
        keep = jnp.concatenate(
            [mrowH[:, tf * BB:(tf + 1) * BB],
             mrowH[:, tb * BB:(tb + 1) * BB]], axis=0)
        h = jnp.where(keep, h_new, h)
        c = jnp.where(keep, c_new, c)
        hid_ref[0:H, tf * BB:(tf + 1) * BB] = h[0:H, :]
        hid_ref[H:2 * H, tb * BB:(tb + 1) * BB] = h[H:2 * H, :]

    emT = jnp.dot(wfc_ref[...], hid_ref[...], preferred_element_type=f32)

    trans3 = crf_ref[0:64, :].reshape(NTP, NTP, BB)
    bos_b = crf_ref[64:72, :]
    stop_b = crf_ref[72:80, :]
    alphas = bos_b + (emT[:, 0:BB] + bfc_b)
    iota_prev = jax.lax.broadcasted_iota(i32, (NTP, NTP, BB), 1)
    iota_tag = jax.lax.broadcasted_iota(i32, (NTP, BB), 0)
    bp = [None] * (T - 1)
    for i in range(1, T):
        scores = trans3 + alphas[None, :, :]
        mx = jnp.max(scores, axis=1)
        bp[i - 1] = jnp.min(
            jnp.where(scores >= mx[:, None, :], iota_prev, NTP), axis=1)
        alphas = jnp.where(mrow8[:, i * BB:(i + 1) * BB],
                           mx + (emT[:, i * BB:(i + 1) * BB] + bfc_b), alphas)

    end_scores = alphas + stop_b
    best = jnp.max(end_scores, axis=0, keepdims=True)
    final_tag = jnp.min(
        jnp.where(end_scores >= best, iota_tag, NTP), axis=0, keepdims=True)

    lengths = jnp.sum(maskb.astype(i32), axis=0, keepdims=True)
    cur = final_tag
    rows = [None] * T
    for t in range(T - 1, -1, -1):
        bp_t = bp[min(t, T - 2)]
        prev = jnp.sum(jnp.where(iota_tag == cur, bp_t, 0), axis=0, keepdims=True)
        cur = jnp.where((lengths - 1) == t, final_tag,
                        jnp.where(t < lengths - 1, prev, cur))
        rows[t] = jnp.where(t < lengths, cur, PAD_ID)

    path_ref[...] = jnp.concatenate(rows, axis=0)
    score_ref[...] = best


def kernel(narrow, wide, x):
    B0, T = x.shape
    BP = ((B0 + BB - 1) // BB) * BB
    nb = BP // BB
    TB = T * BB

    wemb = jnp.zeros((EMB, VPAD2), jnp.float32).at[:, :VPAD].set(
        narrow[N_EMB:N_EMB + VPAD, 0:EMB].T)
    wconv = narrow[N_WCONV:N_WCONV + KMAX * EMB, 0:CH].T
    wx = wide[W_WX:W_WX + CH, :].T
    wh = wide[W_WH:W_WH + 2 * H, :].T
    wfc = narrow[N_WFC:N_WFC + 2 * H, 0:NTP].T
    transT = narrow[N_TRANS:N_TRANS + NTP, 0:NTP]
    crf = jnp.concatenate(
        [jnp.broadcast_to(transT.reshape(64, 1), (64, BB)),
         jnp.broadcast_to(narrow[N_BOS, 0:NTP][:, None], (NTP, BB)),
         jnp.broadcast_to(transT[STOP, :][:, None], (NTP, BB)),
         jnp.broadcast_to(narrow[N_BCONV, 0:CH][:, None], (CH, BB)),
         jnp.broadcast_to(narrow[N_BFC, 0:NTP][:, None], (NTP, BB)),
         jnp.broadcast_to(wide[W_BX, :][:, None], (8 * H, BB))], axis=0)

    xp = jnp.pad(x.astype(jnp.int32), ((0, BP - B0), (0, 0)),
                 constant_values=PAD_ID)
    xT3 = xp.reshape(nb, BB, T).transpose(0, 2, 1)
    xrow3 = xT3.reshape(nb, 1, TB)

    path3, score3 = pl.pallas_call(
        functools.partial(_fused, T=T),
        grid=(nb,),
        in_specs=[
            pl.BlockSpec((None, T, BB), lambda j: (j, 0, 0)),
            pl.BlockSpec((None, 1, TB), lambda j: (j, 0, 0)),
            pl.BlockSpec((EMB, VPAD2), lambda j: (0, 0)),
            pl.BlockSpec((CH, KMAX * EMB), lambda j: (0, 0)),
            pl.BlockSpec((8 * H, CH), lambda j: (0, 0)),
            pl.BlockSpec((8 * H, 2 * H), lambda j: (0, 0)),
            pl.BlockSpec((NTP, 2 * H), lambda j: (0, 0)),
            pl.BlockSpec((376, BB), lambda j: (0, 0)),
        ],
        out_specs=[
            pl.BlockSpec((None, T, BB), lambda j: (j, 0, 0)),
            pl.BlockSpec((None, 1, BB), lambda j: (j, 0, 0)),
        ],
        out_shape=[
            jax.ShapeDtypeStruct((nb, T, BB), jnp.int32),
            jax.ShapeDtypeStruct((nb, 1, BB), jnp.float32),
        ],
        scratch_shapes=[pltpu.VMEM((2 * H, TB), jnp.float32)],
        compiler_params=pltpu.CompilerParams(dimension_semantics=("parallel",)),
    )(xT3, xrow3, wemb, wconv, wx, wh, wfc, crf)

    path = path3.transpose(0, 2, 1).reshape(BP, T)[:B0]
    score = score3.reshape(BP)[:B0]
    return score, path
